# baseline (speedup 1.0000x reference)
import functools

import jax
import jax.numpy as jnp
from jax import lax
from jax.experimental import pallas as pl
from jax.experimental.pallas import tpu as pltpu
from jax.experimental.pallas import tpu_sc as plsc

TOKENS = 8192
D_MODEL = 2048
N_EXPERTS = 16
L = 16
NC = 2
NS = 16
NW = NC * NS
TPW = TOKENS // NW
CH = 16


def _sc_kernel(x_hbm, g_hbm, out_hbm, xv, gv):
    wid = lax.axis_index("s") * NC + lax.axis_index("c")
    base = wid * TPW

    def chunk_body(ci, carry):
        row0 = base + ci * CH
        pltpu.sync_copy(x_hbm.at[pl.ds(row0, CH)], xv)
        pltpu.sync_copy(g_hbm.at[pl.ds(row0, CH)], gv)

        for j in range(CH):
            g = gv[j, :]
            s = g[0]
            for e in range(1, N_EXPERTS):
                s = s + g[e]
            for kk in range(D_MODEL // L):
                xv[j, pl.ds(kk * L, L)] = xv[j, pl.ds(kk * L, L)] * s
        pltpu.sync_copy(xv, out_hbm.at[pl.ds(row0, CH)])
        return carry

    lax.fori_loop(0, TPW // CH, chunk_body, 0)


def kernel(x, gates):
    mesh = plsc.VectorSubcoreMesh(core_axis_name="c", subcore_axis_name="s")
    run = functools.partial(
        pl.kernel,
        mesh=mesh,
        out_type=jax.ShapeDtypeStruct((TOKENS, D_MODEL), jnp.float32),
        scratch_types=[
            pltpu.VMEM((CH, D_MODEL), jnp.float32),
            pltpu.VMEM((CH, N_EXPERTS), jnp.float32),
        ],
    )(_sc_kernel)
    return run(x, gates)


# speedup vs baseline: 3.0560x; 3.0560x over previous
import jax
import jax.numpy as jnp
from jax.experimental import pallas as pl

_BLOCK_TOKENS = 1536


def _row_scale_kernel(x_ref, g_ref, o_ref):
    s = jnp.sum(g_ref[...], axis=1, keepdims=True)
    o_ref[...] = x_ref[...] * s


def kernel(x, gates):
    tokens, d_model = x.shape
    num_experts = gates.shape[1]
    bt = _BLOCK_TOKENS
    grid = (pl.cdiv(tokens, bt),)
    return pl.pallas_call(
        _row_scale_kernel,
        grid=grid,
        in_specs=[
            pl.BlockSpec((bt, d_model), lambda i: (i, 0)),
            pl.BlockSpec((bt, num_experts), lambda i: (i, 0)),
        ],
        out_specs=pl.BlockSpec((bt, d_model), lambda i: (i, 0)),
        out_shape=jax.ShapeDtypeStruct((tokens, d_model), x.dtype),
    )(x, gates)


# speedup vs baseline: 3.0944x; 1.0126x over previous
import jax
import jax.numpy as jnp
from jax.experimental import pallas as pl

_BLOCK_TOKENS = 1792


def _row_scale_kernel(x_ref, g_ref, o_ref):
    s = jnp.sum(g_ref[...], axis=1, keepdims=True)
    o_ref[...] = x_ref[...] * s


def kernel(x, gates):
    tokens, d_model = x.shape
    num_experts = gates.shape[1]
    bt = _BLOCK_TOKENS
    grid = (pl.cdiv(tokens, bt),)
    return pl.pallas_call(
        _row_scale_kernel,
        grid=grid,
        in_specs=[
            pl.BlockSpec((bt, d_model), lambda i: (i, 0)),
            pl.BlockSpec((bt, num_experts), lambda i: (i, 0)),
        ],
        out_specs=pl.BlockSpec((bt, d_model), lambda i: (i, 0)),
        out_shape=jax.ShapeDtypeStruct((tokens, d_model), x.dtype),
    )(x, gates)
